# baseline (speedup 1.0000x reference)
import functools

import jax
import jax.numpy as jnp
from jax import lax
from jax.experimental import pallas as pl
from jax.experimental.pallas import tpu as pltpu
from jax.experimental.pallas import tpu_sc as plsc

NC = 2
NS = 16
NW = NC * NS
L = 16


def _sc_pool(idx_w, emb_one_flat, e2t, *, B, F, K, BPW):
    mesh = plsc.VectorSubcoreMesh(core_axis_name="c", subcore_axis_name="s")

    @functools.partial(
        pl.kernel,
        out_type=(
            jax.ShapeDtypeStruct((K, B), jnp.float32),
            jax.ShapeDtypeStruct((K, B), jnp.float32),
            jax.ShapeDtypeStruct((B,), jnp.float32),
        ),
        mesh=mesh,
        scratch_types=[
            pltpu.VMEM((F, BPW), jnp.int32),
            pltpu.VMEM((K, F, BPW), jnp.float32),
            pltpu.VMEM((F, BPW), jnp.float32),
            pltpu.VMEM((K, BPW), jnp.float32),
            pltpu.VMEM((K, BPW), jnp.float32),
            pltpu.VMEM((BPW,), jnp.float32),
            pltpu.SemaphoreType.DMA,
            pltpu.SemaphoreType.DMA,
        ],
        compiler_params=pltpu.CompilerParams(use_tc_tiling_on_sc=False),
    )
    def k(idx_hbm, emb1_hbm, e2t_hbm, sum_out, sq_out, one_out,
          idx_v, col_v, one_v, sum_v, sq_v, ones_v, sem2, sem1):
        wid = lax.axis_index("s") * NC + lax.axis_index("c")
        base = wid * BPW
        pltpu.sync_copy(idx_hbm.at[wid], idx_v)

        @pl.loop(0, F)
        def _fire(f):
            for kk in range(K):
                pltpu.async_copy(e2t_hbm.at[kk].at[idx_v.at[f]],
                                 col_v.at[kk].at[f], sem2)
            pltpu.async_copy(emb1_hbm.at[idx_v.at[f]], one_v.at[f], sem1)

        @pl.loop(0, F)
        def _drain(f):
            for kk in range(K):
                pltpu.make_async_copy(emb1_hbm.at[pl.ds(0, BPW)],
                                      col_v.at[kk].at[f], sem2).wait()
            pltpu.make_async_copy(emb1_hbm.at[pl.ds(0, BPW)],
                                  one_v.at[f], sem1).wait()

        def body(j, carry):
            kk = lax.shift_right_logical(j, 3)
            g = jnp.bitwise_and(j, 7) * L
            s = col_v[kk, 0, pl.ds(g, L)]
            q = s * s
            for f in range(1, F):
                v = col_v[kk, f, pl.ds(g, L)]
                s = s + v
                q = q + v * v
            sum_v[kk, pl.ds(g, L)] = s
            sq_v[kk, pl.ds(g, L)] = q
            return carry

        lax.fori_loop(0, K * (BPW // L), body, 0, unroll=False)

        for g in range(BPW // L):
            a = one_v[0, pl.ds(g * L, L)]
            for f in range(1, F):
                a = a + one_v[f, pl.ds(g * L, L)]
            ones_v[pl.ds(g * L, L)] = a

        pltpu.sync_copy(sum_v, sum_out.at[:, pl.ds(base, BPW)])
        pltpu.sync_copy(sq_v, sq_out.at[:, pl.ds(base, BPW)])
        pltpu.sync_copy(ones_v, one_out.at[pl.ds(base, BPW)])

    return k(idx_w, emb_one_flat, e2t)


def _tc_combine(sum_kb, sq_kb, one_sum, dense_t, dense_one_row,
                ds2t, zero_bias):
    K, B = sum_kb.shape

    def body(sum_ref, sq_ref, one_ref, dt_ref, d1_ref, ds2t_ref, bias_ref,
             out_ref):
        dt = dt_ref[...]
        ds2t = ds2t_ref[...]
        s = sum_ref[...] + jnp.dot(ds2t, dt, preferred_element_type=jnp.float32)
        q = sq_ref[...] + jnp.dot(ds2t * ds2t, dt * dt,
                                  preferred_element_type=jnp.float32)
        first = one_ref[...] + jnp.dot(d1_ref[...], dt,
                                       preferred_element_type=jnp.float32)
        second = 0.5 * jnp.sum(s * s - q, axis=0, keepdims=True)
        out_ref[...] = jax.nn.sigmoid(first + second + bias_ref[0, 0])

    return pl.pallas_call(
        body,
        out_shape=jax.ShapeDtypeStruct((1, B), jnp.float32),
    )(sum_kb, sq_kb, one_sum.reshape(1, B), dense_t,
      dense_one_row, ds2t, zero_bias.reshape(1, 1))


def kernel(sparse_inputs, dense_inputs, emb_one, emb_second, dense_one,
           dense_second, zero_bias):
    B, F = sparse_inputs.shape
    V, K = emb_second.shape
    BPW = B // NW
    idx_w = sparse_inputs.astype(jnp.int32).reshape(NW, BPW, F).transpose(0, 2, 1)
    out = _tc_combine(
        *_sc_pool(idx_w, emb_one.reshape(V), emb_second.T,
                  B=B, F=F, K=K, BPW=BPW),
        dense_inputs.T, dense_one.reshape(1, -1),
        dense_second.reshape(-1, K).T, zero_bias)
    return out.reshape(B, 1)


# speedup vs baseline: 2.8787x; 2.8787x over previous
import functools

import jax
import jax.numpy as jnp
from jax import lax
from jax.experimental import pallas as pl
from jax.experimental.pallas import tpu as pltpu
from jax.experimental.pallas import tpu_sc as plsc

NC = 2
NS = 16
NW = NC * NS
L = 16


def _sc_pool(idx_w, emb_one_flat, emb_second_row, *, B, F, K, BPW):
    mesh = plsc.VectorSubcoreMesh(core_axis_name="c", subcore_axis_name="s")

    @functools.partial(
        pl.kernel,
        out_type=(
            jax.ShapeDtypeStruct((B, K), jnp.float32),
            jax.ShapeDtypeStruct((B, K), jnp.float32),
            jax.ShapeDtypeStruct((B,), jnp.float32),
        ),
        mesh=mesh,
        scratch_types=[
            pltpu.VMEM((F, BPW), jnp.int32),
            pltpu.VMEM((F, BPW, K), jnp.float32),
            pltpu.VMEM((F, BPW), jnp.float32),
            pltpu.VMEM((BPW, K), jnp.float32),
            pltpu.VMEM((BPW, K), jnp.float32),
            pltpu.VMEM((BPW,), jnp.float32),
            pltpu.SemaphoreType.DMA,
            pltpu.SemaphoreType.DMA,
        ],
        compiler_params=pltpu.CompilerParams(use_tc_tiling_on_sc=False),
    )
    def k(idx_hbm, emb1_hbm, emb2_hbm, sum_out, sq_out, one_out,
          idx_v, rows_v, one_v, sum_v, sq_v, ones_v, sem2, sem1):
        wid = lax.axis_index("s") * NC + lax.axis_index("c")
        base = wid * BPW
        pltpu.sync_copy(idx_hbm.at[wid], idx_v)
        cps = []
        for f in range(F):
            cps.append(pltpu.async_copy(emb2_hbm.at[idx_v.at[f]], rows_v.at[f], sem2))
            cps.append(pltpu.async_copy(emb1_hbm.at[idx_v.at[f]], one_v.at[f], sem1))
        for c in cps:
            c.wait()

        def body(j, carry):
            s = rows_v[0, j]
            q = s * s
            for f in range(1, F):
                v = rows_v[f, j]
                s = s + v
                q = q + v * v
            sum_v[j] = s
            sq_v[j] = q
            return carry

        lax.fori_loop(0, BPW, body, 0, unroll=False)

        for g in range(BPW // L):
            a = one_v[0, pl.ds(g * L, L)]
            for f in range(1, F):
                a = a + one_v[f, pl.ds(g * L, L)]
            ones_v[pl.ds(g * L, L)] = a

        pltpu.sync_copy(sum_v, sum_out.at[pl.ds(base, BPW)])
        pltpu.sync_copy(sq_v, sq_out.at[pl.ds(base, BPW)])
        pltpu.sync_copy(ones_v, one_out.at[pl.ds(base, BPW)])

    return k(idx_w, emb_one_flat, emb_second_row)


def _tc_combine(sum_vec, sq_vec, one_sum, dense_inputs, dense_one_row,
                dense_second_mat, zero_bias):
    B, K = sum_vec.shape

    def body(sum_ref, sq_ref, one_ref, dense_ref, d1_ref, ds2_ref, bias_ref,
             out_ref):
        dense = dense_ref[...]
        ds2 = ds2_ref[...]
        s = sum_ref[...] + jnp.dot(dense, ds2, preferred_element_type=jnp.float32)
        q = sq_ref[...] + jnp.dot(dense * dense, ds2 * ds2,
                                  preferred_element_type=jnp.float32)
        first = one_ref[...] + jnp.sum(dense * d1_ref[...], axis=1, keepdims=True)
        second = 0.5 * jnp.sum(s * s - q, axis=1, keepdims=True)
        out_ref[...] = jax.nn.sigmoid(first + second + bias_ref[0, 0])

    return pl.pallas_call(
        body,
        out_shape=jax.ShapeDtypeStruct((B, 1), jnp.float32),
    )(sum_vec, sq_vec, one_sum.reshape(B, 1), dense_inputs,
      dense_one_row, dense_second_mat, zero_bias.reshape(1, 1))


def kernel(sparse_inputs, dense_inputs, emb_one, emb_second, dense_one,
           dense_second, zero_bias):
    B, F = sparse_inputs.shape
    V, K = emb_second.shape
    BPW = B // NW
    idx_w = sparse_inputs.astype(jnp.int32).reshape(NW, BPW, F).transpose(0, 2, 1)
    sum_vec, sq_vec, one_sum = _sc_pool(
        idx_w, emb_one.reshape(V), emb_second, B=B, F=F, K=K, BPW=BPW)
    return _tc_combine(sum_vec, sq_vec, one_sum, dense_inputs,
                       dense_one.reshape(1, -1), dense_second.reshape(-1, K),
                       zero_bias)
